# Initial kernel scaffold; baseline (speedup 1.0000x reference)
import jax
import jax.numpy as jnp
from jax.experimental import pallas as pl


def kernel(c, h_pl, h_mi, edge_index, edge_weight, W, b):
    raise NotImplementedError("write your pallas kernel here")


# speedup vs baseline: 20.4933x; 20.4933x over previous
import functools

import jax
import jax.numpy as jnp
from jax import lax
from jax.experimental import pallas as pl
from jax.experimental.pallas import tpu as pltpu
from jax.experimental.pallas import tpu_sc as plsc

_NUM_CORES = 2
_NUM_SUBCORES = 16
_NW = _NUM_CORES * _NUM_SUBCORES
_IDX_BLK = 128
_LANES = 16


def _dense_body(c_ref, w_ref, hp_ref, hm_ref, zp_ref, zm_ref):
    c_row = c_ref[...]
    u_row = lax.dot_general(c_row, w_ref[...], (((1,), (1,)), ((), ())),
                            preferred_element_type=jnp.float32)
    m_t = jnp.concatenate([c_row, u_row], axis=0)
    zp_ref[...] = lax.dot_general(hp_ref[...], m_t, (((1,), (1,)), ((), ())),
                                  preferred_element_type=jnp.float32)
    zm_ref[...] = lax.dot_general(hm_ref[...], m_t, (((1,), (1,)), ((), ())),
                                  preferred_element_type=jnp.float32)


def _sc_body(nblk, p_hbm, q_hbm, col_hbm, row_hbm, ew_hbm, zero_hbm,
             part1_hbm, part2_hbm,
             col_v, row_v, ew_v, pv, qv, v1, v2, acc1, acc2, sem_p, sem_q):
    cid = lax.axis_index("c")
    sid = lax.axis_index("s")
    wid = cid * _NUM_SUBCORES + sid

    pltpu.sync_copy(col_hbm.at[wid], col_v)
    pltpu.sync_copy(row_hbm.at[wid], row_v)
    pltpu.sync_copy(ew_hbm.at[wid], ew_v)

    @pl.when(sid == 0)
    def _zero():
        pltpu.sync_copy(zero_hbm, acc1)
        pltpu.sync_copy(zero_hbm, acc2)

    plsc.subcore_barrier()

    def _blk(j, carry):
        base = pl.multiple_of(j * _IDX_BLK, _IDX_BLK)
        dst_p = pv.at[pl.ds(base, _IDX_BLK)]
        dst_q = qv.at[pl.ds(base, _IDX_BLK)]
        cp_p = pltpu.async_copy(p_hbm.at[col_v.at[j]], dst_p, sem_p)
        cp_q = pltpu.async_copy(q_hbm.at[col_v.at[j]], dst_q, sem_q)
        cp_p.wait()
        cp_q.wait()
        for k in range(_IDX_BLK // _LANES):
            sl = pl.ds(base + k * _LANES, _LANES)
            v1[sl] = ew_v[sl] * pv[sl]
            v2[sl] = ew_v[sl] * qv[sl]
        pltpu.sync_copy(v1.at[pl.ds(base, _IDX_BLK)], acc1.at[row_v.at[j]], add=True)
        pltpu.sync_copy(v2.at[pl.ds(base, _IDX_BLK)], acc2.at[row_v.at[j]], add=True)
        return carry

    lax.fori_loop(0, nblk, _blk, 0)
    plsc.subcore_barrier()

    @pl.when(sid == 0)
    def _flush():
        pltpu.sync_copy(acc1, part1_hbm.at[cid])
        pltpu.sync_copy(acc2, part2_hbm.at[cid])


def _combine_body(b_ref, zp_ref, zm_ref, p1_ref, p2_ref, out_ref):
    n = zp_ref.shape[0]
    bval = b_ref[0]
    out_ref[0, pl.ds(0 * n, n)] = zp_ref[:, 1] + bval
    out_ref[0, pl.ds(1 * n, n)] = p1_ref[0, :] + p1_ref[1, :]
    out_ref[0, pl.ds(2 * n, n)] = zm_ref[:, 1] + bval
    out_ref[0, pl.ds(3 * n, n)] = p2_ref[0, :] + p2_ref[1, :]


def _row_block(n):
    for cand in (1024, 1000, 512, 500, 256, 200, 128, 8):
        if n % cand == 0 and cand % 8 == 0:
            return cand
    return n


def kernel(c, h_pl, h_mi, edge_index, edge_weight, W, b):
    n = h_pl.shape[1]
    n_h = h_pl.shape[2]
    e = edge_weight.shape[0]

    hp = h_pl.reshape(n, n_h)
    hm = h_mi.reshape(n, n_h)
    w2 = W.reshape(n_h, n_h)

    br = _row_block(n)
    zp, zm = pl.pallas_call(
        _dense_body,
        grid=(n // br,),
        in_specs=[
            pl.BlockSpec((1, n_h), lambda i: (0, 0)),
            pl.BlockSpec((n_h, n_h), lambda i: (0, 0)),
            pl.BlockSpec((br, n_h), lambda i: (i, 0)),
            pl.BlockSpec((br, n_h), lambda i: (i, 0)),
        ],
        out_specs=[
            pl.BlockSpec((br, 2), lambda i: (i, 0)),
            pl.BlockSpec((br, 2), lambda i: (i, 0)),
        ],
        out_shape=[
            jax.ShapeDtypeStruct((n, 2), jnp.float32),
            jax.ShapeDtypeStruct((n, 2), jnp.float32),
        ],
    )(c, w2, hp, hm)

    p = zp[:, 0]
    q = zm[:, 0]

    nblk = -(-e // (_NW * _IDX_BLK))
    per_w = nblk * _IDX_BLK
    pad = _NW * per_w - e
    col = jnp.concatenate([edge_index[1], jnp.zeros((pad,), jnp.int32)])
    row = jnp.concatenate([edge_index[0], jnp.zeros((pad,), jnp.int32)])
    ew = jnp.concatenate([edge_weight, jnp.zeros((pad,), jnp.float32)])
    col = col.reshape(_NW, nblk, _IDX_BLK)
    row = row.reshape(_NW, nblk, _IDX_BLK)
    ew = ew.reshape(_NW, per_w)
    zero = jnp.zeros((n,), jnp.float32)

    sc = pl.kernel(
        functools.partial(_sc_body, nblk),
        out_type=[jax.ShapeDtypeStruct((_NUM_CORES, n), jnp.float32),
                  jax.ShapeDtypeStruct((_NUM_CORES, n), jnp.float32)],
        mesh=plsc.VectorSubcoreMesh(core_axis_name="c", subcore_axis_name="s"),
        scratch_types=[
            pltpu.VMEM((nblk, _IDX_BLK), jnp.int32),
            pltpu.VMEM((nblk, _IDX_BLK), jnp.int32),
            pltpu.VMEM((per_w,), jnp.float32),
            pltpu.VMEM((per_w,), jnp.float32),
            pltpu.VMEM((per_w,), jnp.float32),
            pltpu.VMEM((per_w,), jnp.float32),
            pltpu.VMEM((per_w,), jnp.float32),
            pltpu.VMEM_SHARED((n,), jnp.float32),
            pltpu.VMEM_SHARED((n,), jnp.float32),
            pltpu.SemaphoreType.DMA,
            pltpu.SemaphoreType.DMA,
        ],
    )
    part1, part2 = sc(p, q, col, row, ew, zero)

    logits = pl.pallas_call(
        _combine_body,
        in_specs=[
            pl.BlockSpec(memory_space=pltpu.SMEM),
            pl.BlockSpec(memory_space=pltpu.VMEM),
            pl.BlockSpec(memory_space=pltpu.VMEM),
            pl.BlockSpec(memory_space=pltpu.VMEM),
            pl.BlockSpec(memory_space=pltpu.VMEM),
        ],
        out_specs=pl.BlockSpec(memory_space=pltpu.VMEM),
        out_shape=jax.ShapeDtypeStruct((1, 4 * n), jnp.float32),
    )(b, zp, zm, part1, part2)
    return logits


# speedup vs baseline: 22.5696x; 1.1013x over previous
import functools

import jax
import jax.numpy as jnp
from jax import lax
from jax.experimental import pallas as pl
from jax.experimental.pallas import tpu as pltpu
from jax.experimental.pallas import tpu_sc as plsc

_NUM_CORES = 2
_NUM_SUBCORES = 16
_NW = _NUM_CORES * _NUM_SUBCORES
_IDX_BLK = 128
_LANES = 16


def _dense_body(c_ref, w_ref, hp_ref, hm_ref, zp_ref, zm_ref):
    c_row = c_ref[...]
    u_row = lax.dot_general(c_row, w_ref[...], (((1,), (1,)), ((), ())),
                            preferred_element_type=jnp.float32)
    m_t = jnp.concatenate([c_row, u_row], axis=0)
    zp_ref[...] = lax.dot_general(hp_ref[...], m_t, (((1,), (1,)), ((), ())),
                                  preferred_element_type=jnp.float32)
    zm_ref[...] = lax.dot_general(hm_ref[...], m_t, (((1,), (1,)), ((), ())),
                                  preferred_element_type=jnp.float32)


def _sc_body(nblk, p_hbm, q_hbm, col_hbm, row_hbm, ew_hbm, zero_hbm,
             part1_hbm, part2_hbm,
             col_v, row_v, ew_v, pv, qv, v1, v2, acc1, acc2,
             sem_s, sem_p, sem_q):
    cid = lax.axis_index("c")
    sid = lax.axis_index("s")
    wid = cid * _NUM_SUBCORES + sid

    cp_c = pltpu.async_copy(col_hbm.at[wid], col_v, sem_s)
    cp_r = pltpu.async_copy(row_hbm.at[wid], row_v, sem_s)
    cp_w = pltpu.async_copy(ew_hbm.at[wid], ew_v, sem_s)

    @pl.when(sid == 0)
    def _zero():
        pltpu.sync_copy(zero_hbm, acc1)
        pltpu.sync_copy(zero_hbm, acc2)

    cp_c.wait()
    cp_r.wait()
    cp_w.wait()

    cp_p = pltpu.async_copy(p_hbm.at[col_v], pv, sem_p)
    cp_q = pltpu.async_copy(q_hbm.at[col_v], qv, sem_q)
    plsc.subcore_barrier()
    cp_p.wait()
    cp_q.wait()

    def _blk(j, carry):
        base = pl.multiple_of(j * _IDX_BLK, _IDX_BLK)
        for k in range(_IDX_BLK // _LANES):
            sl = pl.ds(base + k * _LANES, _LANES)
            v1[sl] = ew_v[sl] * pv[sl]
            v2[sl] = ew_v[sl] * qv[sl]
        return carry

    lax.fori_loop(0, nblk, _blk, 0)
    pltpu.sync_copy(v1, acc1.at[row_v], add=True)
    pltpu.sync_copy(v2, acc2.at[row_v], add=True)
    plsc.subcore_barrier()

    @pl.when(sid == 0)
    def _flush():
        pltpu.sync_copy(acc1, part1_hbm.at[cid])
        pltpu.sync_copy(acc2, part2_hbm.at[cid])


def _combine_body(b_ref, zp_ref, zm_ref, p1_ref, p2_ref, out_ref):
    n = zp_ref.shape[0]
    bval = b_ref[0]
    out_ref[0, pl.ds(0 * n, n)] = zp_ref[:, 1] + bval
    out_ref[0, pl.ds(1 * n, n)] = p1_ref[0, :] + p1_ref[1, :]
    out_ref[0, pl.ds(2 * n, n)] = zm_ref[:, 1] + bval
    out_ref[0, pl.ds(3 * n, n)] = p2_ref[0, :] + p2_ref[1, :]


def _row_block(n):
    for cand in (1024, 1000, 512, 500, 256, 200, 128, 8):
        if n % cand == 0 and cand % 8 == 0:
            return cand
    return n


def kernel(c, h_pl, h_mi, edge_index, edge_weight, W, b):
    n = h_pl.shape[1]
    n_h = h_pl.shape[2]
    e = edge_weight.shape[0]

    hp = h_pl.reshape(n, n_h)
    hm = h_mi.reshape(n, n_h)
    w2 = W.reshape(n_h, n_h)

    br = _row_block(n)
    zp, zm = pl.pallas_call(
        _dense_body,
        grid=(n // br,),
        in_specs=[
            pl.BlockSpec((1, n_h), lambda i: (0, 0)),
            pl.BlockSpec((n_h, n_h), lambda i: (0, 0)),
            pl.BlockSpec((br, n_h), lambda i: (i, 0)),
            pl.BlockSpec((br, n_h), lambda i: (i, 0)),
        ],
        out_specs=[
            pl.BlockSpec((br, 2), lambda i: (i, 0)),
            pl.BlockSpec((br, 2), lambda i: (i, 0)),
        ],
        out_shape=[
            jax.ShapeDtypeStruct((n, 2), jnp.float32),
            jax.ShapeDtypeStruct((n, 2), jnp.float32),
        ],
    )(c, w2, hp, hm)

    p = zp[:, 0]
    q = zm[:, 0]

    nblk = -(-e // (_NW * _IDX_BLK))
    per_w = nblk * _IDX_BLK
    pad = _NW * per_w - e
    col = jnp.concatenate([edge_index[1], jnp.zeros((pad,), jnp.int32)])
    row = jnp.concatenate([edge_index[0], jnp.zeros((pad,), jnp.int32)])
    ew = jnp.concatenate([edge_weight, jnp.zeros((pad,), jnp.float32)])
    col = col.reshape(_NW, per_w)
    row = row.reshape(_NW, per_w)
    ew = ew.reshape(_NW, per_w)
    zero = jnp.zeros((n,), jnp.float32)

    sc = pl.kernel(
        functools.partial(_sc_body, nblk),
        out_type=[jax.ShapeDtypeStruct((_NUM_CORES, n), jnp.float32),
                  jax.ShapeDtypeStruct((_NUM_CORES, n), jnp.float32)],
        mesh=plsc.VectorSubcoreMesh(core_axis_name="c", subcore_axis_name="s"),
        scratch_types=[
            pltpu.VMEM((per_w,), jnp.int32),
            pltpu.VMEM((per_w,), jnp.int32),
            pltpu.VMEM((per_w,), jnp.float32),
            pltpu.VMEM((per_w,), jnp.float32),
            pltpu.VMEM((per_w,), jnp.float32),
            pltpu.VMEM((per_w,), jnp.float32),
            pltpu.VMEM((per_w,), jnp.float32),
            pltpu.VMEM_SHARED((n,), jnp.float32),
            pltpu.VMEM_SHARED((n,), jnp.float32),
            pltpu.SemaphoreType.DMA,
            pltpu.SemaphoreType.DMA,
            pltpu.SemaphoreType.DMA,
        ],
    )
    part1, part2 = sc(p, q, col, row, ew, zero)

    logits = pl.pallas_call(
        _combine_body,
        in_specs=[
            pl.BlockSpec(memory_space=pltpu.SMEM),
            pl.BlockSpec(memory_space=pltpu.VMEM),
            pl.BlockSpec(memory_space=pltpu.VMEM),
            pl.BlockSpec(memory_space=pltpu.VMEM),
            pl.BlockSpec(memory_space=pltpu.VMEM),
        ],
        out_specs=pl.BlockSpec(memory_space=pltpu.VMEM),
        out_shape=jax.ShapeDtypeStruct((1, 4 * n), jnp.float32),
    )(b, zp, zm, part1, part2)
    return logits


# speedup vs baseline: 51.1759x; 2.2675x over previous
import functools

import jax
import jax.numpy as jnp
from jax import lax
from jax.experimental import pallas as pl
from jax.experimental.pallas import tpu as pltpu
from jax.experimental.pallas import tpu_sc as plsc

_NUM_CORES = 2
_NUM_SUBCORES = 16
_NW = _NUM_CORES * _NUM_SUBCORES
_LANES = 16


def _dense_body(c_ref, w_ref, hp_ref, hm_ref, p_ref, q_ref, s1_ref, s2_ref):
    c_row = c_ref[...]
    u_row = lax.dot_general(c_row, w_ref[...], (((1,), (1,)), ((), ())),
                            preferred_element_type=jnp.float32)
    m_t = jnp.concatenate([c_row, u_row], axis=0)
    zp = lax.dot_general(m_t, hp_ref[...], (((1,), (1,)), ((), ())),
                         preferred_element_type=jnp.float32)
    zm = lax.dot_general(m_t, hm_ref[...], (((1,), (1,)), ((), ())),
                         preferred_element_type=jnp.float32)
    p_ref[...] = zp[0, :]
    s1_ref[...] = zp[1, :]
    q_ref[...] = zm[0, :]
    s2_ref[...] = zm[1, :]


def _sc_body(per_w, p_hbm, q_hbm, colg_hbm, rowg_hbm, ew_hbm, zero_hbm,
             part1_hbm, part2_hbm,
             col_v, row_v, ew_v, pv, qv, v1, v2, p_s, q_s, acc1, acc2,
             sem_s, sem_p, sem_q):
    cid = lax.axis_index("c")
    sid = lax.axis_index("s")
    wid = cid * _NUM_SUBCORES + sid
    base = wid * per_w

    cp_c = pltpu.async_copy(colg_hbm.at[pl.ds(base, per_w)], col_v, sem_s)
    cp_r = pltpu.async_copy(rowg_hbm.at[pl.ds(base, per_w)], row_v, sem_s)
    cp_w = pltpu.async_copy(ew_hbm.at[pl.ds(base, per_w)], ew_v, sem_s)

    @pl.when(sid == 0)
    def _init0():
        pltpu.sync_copy(zero_hbm, acc1)
        pltpu.sync_copy(p_hbm, p_s)

    @pl.when(sid == 1)
    def _init1():
        pltpu.sync_copy(zero_hbm, acc2)
        pltpu.sync_copy(q_hbm, q_s)

    cp_c.wait()
    cp_r.wait()
    cp_w.wait()
    plsc.subcore_barrier()

    cp_p = pltpu.async_copy(p_s.at[col_v], pv, sem_p)
    cp_q = pltpu.async_copy(q_s.at[col_v], qv, sem_q)
    cp_p.wait()
    cp_q.wait()

    nfull = per_w // _LANES

    def _vec(j, carry):
        sl = pl.ds(pl.multiple_of(j * _LANES, _LANES), _LANES)
        v1[sl] = ew_v[sl] * pv[sl]
        v2[sl] = ew_v[sl] * qv[sl]
        return carry

    lax.fori_loop(0, nfull, _vec, 0)
    if per_w % _LANES:
        sl = pl.ds(per_w - _LANES, _LANES)
        v1[sl] = ew_v[sl] * pv[sl]
        v2[sl] = ew_v[sl] * qv[sl]

    pltpu.sync_copy(v1, acc1.at[row_v], add=True)
    pltpu.sync_copy(v2, acc2.at[row_v], add=True)
    plsc.subcore_barrier()

    @pl.when(sid == 0)
    def _flush0():
        pltpu.sync_copy(acc1, part1_hbm.at[cid])

    @pl.when(sid == 1)
    def _flush1():
        pltpu.sync_copy(acc2, part2_hbm.at[cid])


def _combine_body(b_ref, s1_ref, s2_ref, p1_ref, p2_ref, out_ref):
    n = s1_ref.shape[0]
    bval = b_ref[0]
    out_ref[0, pl.ds(0, n)] = s1_ref[...] + bval
    out_ref[0, pl.ds(n, n)] = p1_ref[0, :] + p1_ref[1, :]
    out_ref[0, pl.ds(2 * n, n)] = s2_ref[...] + bval
    out_ref[0, pl.ds(3 * n, n)] = p2_ref[0, :] + p2_ref[1, :]


def kernel(c, h_pl, h_mi, edge_index, edge_weight, W, b):
    n = h_pl.shape[1]
    n_h = h_pl.shape[2]
    e = edge_weight.shape[0]

    hp = h_pl.reshape(n, n_h)
    hm = h_mi.reshape(n, n_h)
    w2 = W.reshape(n_h, n_h)

    p, q, s1, s2 = pl.pallas_call(
        _dense_body,
        out_shape=[
            jax.ShapeDtypeStruct((n,), jnp.float32),
            jax.ShapeDtypeStruct((n,), jnp.float32),
            jax.ShapeDtypeStruct((n,), jnp.float32),
            jax.ShapeDtypeStruct((n,), jnp.float32),
        ],
    )(c, w2, hp, hm)

    align = _NW * 8
    if e % align == 0:
        colg = edge_index[1]
        rowg = edge_index[0]
        ew = edge_weight
        e_pad = e
    else:
        e_pad = -(-e // align) * align
        pad = e_pad - e
        ei = jnp.concatenate(
            [edge_index, jnp.zeros((2, pad), edge_index.dtype)], axis=1)
        colg = ei[1]
        rowg = ei[0]
        ew = jnp.concatenate([edge_weight, jnp.zeros((pad,), jnp.float32)])
    per_w = e_pad // _NW
    zero = jnp.zeros((n,), jnp.float32)

    sc = pl.kernel(
        functools.partial(_sc_body, per_w),
        out_type=[jax.ShapeDtypeStruct((_NUM_CORES, n), jnp.float32),
                  jax.ShapeDtypeStruct((_NUM_CORES, n), jnp.float32)],
        mesh=plsc.VectorSubcoreMesh(core_axis_name="c", subcore_axis_name="s"),
        scratch_types=[
            pltpu.VMEM((per_w,), jnp.int32),
            pltpu.VMEM((per_w,), jnp.int32),
            pltpu.VMEM((per_w,), jnp.float32),
            pltpu.VMEM((per_w,), jnp.float32),
            pltpu.VMEM((per_w,), jnp.float32),
            pltpu.VMEM((per_w,), jnp.float32),
            pltpu.VMEM((per_w,), jnp.float32),
            pltpu.VMEM_SHARED((n,), jnp.float32),
            pltpu.VMEM_SHARED((n,), jnp.float32),
            pltpu.VMEM_SHARED((n,), jnp.float32),
            pltpu.VMEM_SHARED((n,), jnp.float32),
            pltpu.SemaphoreType.DMA,
            pltpu.SemaphoreType.DMA,
            pltpu.SemaphoreType.DMA,
        ],
    )
    part1, part2 = sc(p, q, colg, rowg, ew, zero)

    logits = pl.pallas_call(
        _combine_body,
        in_specs=[
            pl.BlockSpec(memory_space=pltpu.SMEM),
            pl.BlockSpec(memory_space=pltpu.VMEM),
            pl.BlockSpec(memory_space=pltpu.VMEM),
            pl.BlockSpec(memory_space=pltpu.VMEM),
            pl.BlockSpec(memory_space=pltpu.VMEM),
        ],
        out_specs=pl.BlockSpec(memory_space=pltpu.VMEM),
        out_shape=jax.ShapeDtypeStruct((1, 4 * n), jnp.float32),
    )(b, s1, s2, part1, part2)
    return logits


# speedup vs baseline: 59.4599x; 1.1619x over previous
import functools

import jax
import jax.numpy as jnp
from jax import lax
from jax.experimental import pallas as pl
from jax.experimental.pallas import tpu as pltpu
from jax.experimental.pallas import tpu_sc as plsc

_NUM_CORES = 2
_NUM_SUBCORES = 16
_NW = _NUM_CORES * _NUM_SUBCORES
_LANES = 16


def _dense_body(c_ref, w_ref, hp_ref, hm_ref, p_ref, q_ref, s1_ref, s2_ref):
    c_row = c_ref[...]
    u_row = lax.dot_general(c_row, w_ref[...], (((1,), (1,)), ((), ())),
                            preferred_element_type=jnp.float32)
    m_t = jnp.concatenate([c_row, u_row], axis=0)
    zp = lax.dot_general(m_t, hp_ref[...], (((1,), (1,)), ((), ())),
                         preferred_element_type=jnp.float32)
    zm = lax.dot_general(m_t, hm_ref[...], (((1,), (1,)), ((), ())),
                         preferred_element_type=jnp.float32)
    p_ref[...] = zp[0, :]
    s1_ref[...] = zp[1, :]
    q_ref[...] = zm[0, :]
    s2_ref[...] = zm[1, :]


def _sc_body(chunk, e, p_hbm, q_hbm, ei_hbm, ew_hbm, zero_hbm,
             part1_hbm, part2_hbm,
             edg_v, col_v, row_v, ew_v, pv, qv, v1, v2, p_s, q_s, acc1, acc2,
             sem_s, sem_p, sem_q):
    cid = lax.axis_index("c")
    sid = lax.axis_index("s")
    wid = cid * _NUM_SUBCORES + sid
    lo = wid * chunk
    hi = jnp.minimum(lo + chunk, e)
    start = pl.multiple_of(jnp.minimum(lo, e - chunk), 128)

    cp_e = pltpu.async_copy(ei_hbm.at[:, pl.ds(start, chunk)], edg_v, sem_s)
    cp_w = pltpu.async_copy(ew_hbm.at[pl.ds(start, chunk)], ew_v, sem_s)

    @pl.when(sid == 0)
    def _init0():
        pltpu.sync_copy(zero_hbm, acc1)
        pltpu.sync_copy(p_hbm, p_s)

    @pl.when(sid == 1)
    def _init1():
        pltpu.sync_copy(zero_hbm, acc2)
        pltpu.sync_copy(q_hbm, q_s)

    cp_e.wait()
    cp_w.wait()
    def _cpy(j, carry):
        sl = pl.ds(pl.multiple_of(j * _LANES, _LANES), _LANES)
        row_v[sl] = edg_v[0, sl]
        col_v[sl] = edg_v[1, sl]
        return carry

    lax.fori_loop(0, chunk // _LANES, _cpy, 0)
    plsc.subcore_barrier()

    cp_p = pltpu.async_copy(p_s.at[col_v], pv, sem_p)
    cp_q = pltpu.async_copy(q_s.at[col_v], qv, sem_q)
    cp_p.wait()
    cp_q.wait()

    lanes = lax.broadcasted_iota(jnp.int32, (_LANES,), 0)

    def _vec(j, carry):
        sl = pl.ds(pl.multiple_of(j * _LANES, _LANES), _LANES)
        gidx = start + j * _LANES + lanes
        wv = jnp.where((gidx >= lo) & (gidx < hi), ew_v[sl], 0.0)
        v1[sl] = wv * pv[sl]
        v2[sl] = wv * qv[sl]
        return carry

    lax.fori_loop(0, chunk // _LANES, _vec, 0)

    pltpu.sync_copy(v1, acc1.at[row_v], add=True)
    pltpu.sync_copy(v2, acc2.at[row_v], add=True)
    plsc.subcore_barrier()

    @pl.when(sid == 0)
    def _flush0():
        pltpu.sync_copy(acc1, part1_hbm.at[cid])

    @pl.when(sid == 1)
    def _flush1():
        pltpu.sync_copy(acc2, part2_hbm.at[cid])


def _combine_body(b_ref, s1_ref, s2_ref, p1_ref, p2_ref, out_ref):
    n = s1_ref.shape[0]
    bval = b_ref[0]
    out_ref[0, pl.ds(0, n)] = s1_ref[...] + bval
    out_ref[0, pl.ds(n, n)] = p1_ref[0, :] + p1_ref[1, :]
    out_ref[0, pl.ds(2 * n, n)] = s2_ref[...] + bval
    out_ref[0, pl.ds(3 * n, n)] = p2_ref[0, :] + p2_ref[1, :]


def kernel(c, h_pl, h_mi, edge_index, edge_weight, W, b):
    n = h_pl.shape[1]
    n_h = h_pl.shape[2]
    e = edge_weight.shape[0]

    hp = h_pl.reshape(n, n_h)
    hm = h_mi.reshape(n, n_h)
    w2 = W.reshape(n_h, n_h)

    p, q, s1, s2 = pl.pallas_call(
        _dense_body,
        out_shape=[
            jax.ShapeDtypeStruct((n,), jnp.float32),
            jax.ShapeDtypeStruct((n,), jnp.float32),
            jax.ShapeDtypeStruct((n,), jnp.float32),
            jax.ShapeDtypeStruct((n,), jnp.float32),
        ],
    )(c, w2, hp, hm)

    if e % 128 == 0:
        ei = edge_index
        ew = edge_weight
        e_pad = e
    else:
        e_pad = -(-e // 128) * 128
        pad = e_pad - e
        ei = jnp.concatenate(
            [edge_index, jnp.zeros((2, pad), edge_index.dtype)], axis=1)
        ew = jnp.concatenate([edge_weight, jnp.zeros((pad,), jnp.float32)])
    chunk = -(-e_pad // (_NW * 128)) * 128
    zero = jnp.zeros((n,), jnp.float32)

    sc = pl.kernel(
        functools.partial(_sc_body, chunk, e_pad),
        out_type=[jax.ShapeDtypeStruct((_NUM_CORES, n), jnp.float32),
                  jax.ShapeDtypeStruct((_NUM_CORES, n), jnp.float32)],
        mesh=plsc.VectorSubcoreMesh(core_axis_name="c", subcore_axis_name="s"),
        scratch_types=[
            pltpu.VMEM((2, chunk), jnp.int32),
            pltpu.VMEM((chunk,), jnp.int32),
            pltpu.VMEM((chunk,), jnp.int32),
            pltpu.VMEM((chunk,), jnp.float32),
            pltpu.VMEM((chunk,), jnp.float32),
            pltpu.VMEM((chunk,), jnp.float32),
            pltpu.VMEM((chunk,), jnp.float32),
            pltpu.VMEM((chunk,), jnp.float32),
            pltpu.VMEM_SHARED((n,), jnp.float32),
            pltpu.VMEM_SHARED((n,), jnp.float32),
            pltpu.VMEM_SHARED((n,), jnp.float32),
            pltpu.VMEM_SHARED((n,), jnp.float32),
            pltpu.SemaphoreType.DMA,
            pltpu.SemaphoreType.DMA,
            pltpu.SemaphoreType.DMA,
        ],
    )
    part1, part2 = sc(p, q, ei, ew, zero)

    logits = pl.pallas_call(
        _combine_body,
        in_specs=[
            pl.BlockSpec(memory_space=pltpu.SMEM),
            pl.BlockSpec(memory_space=pltpu.VMEM),
            pl.BlockSpec(memory_space=pltpu.VMEM),
            pl.BlockSpec(memory_space=pltpu.VMEM),
            pl.BlockSpec(memory_space=pltpu.VMEM),
        ],
        out_specs=pl.BlockSpec(memory_space=pltpu.VMEM),
        out_shape=jax.ShapeDtypeStruct((1, 4 * n), jnp.float32),
    )(b, s1, s2, part1, part2)
    return logits
